# Initial kernel scaffold; baseline (speedup 1.0000x reference)
import jax
import jax.numpy as jnp
from jax.experimental import pallas as pl


def kernel(x, W1, att_src1, att_dst1, b1, W2, att_src2, att_dst2, b2):
    raise NotImplementedError("write your pallas kernel here")


# speedup vs baseline: 27.5580x; 27.5580x over previous
import functools

import jax
import jax.numpy as jnp
from jax.experimental import pallas as pl

B_ = 2
N_ = 20
C_ = 256
M_ = 250
K_ = 16
H_ = 256
P_ = N_ * M_
PP = 5120
R_ = 512
CB = 512


def _mask_kernel(nodes_ref, nodes_t_ref, mask_ref):
    nb = nodes_ref[0]
    nt = nodes_t_ref[0]
    g = jnp.dot(nb, nt, preferred_element_type=jnp.float32)
    sqr = jnp.sum(nb * nb, axis=1, keepdims=True)
    sqc = jnp.sum(nt * nt, axis=0, keepdims=True)
    d2 = sqr + sqc - 2.0 * g
    r = nb.shape[0]
    col = jax.lax.broadcasted_iota(jnp.int32, (r, PP), 1)
    d2 = jnp.where(col < P_, d2, jnp.inf)
    v = d2
    for _ in range(K_):
        m = jnp.min(v, axis=1, keepdims=True)
        v = jnp.where(v <= m, jnp.inf, v)
    t = jnp.min(v, axis=1, keepdims=True)
    row = jax.lax.broadcasted_iota(jnp.int32, (r, PP), 0) + pl.program_id(1) * r
    mask = (d2 <= t) & (col < P_) & (row < P_)
    mask_ref[0] = mask.astype(mask_ref.dtype)


def _proj_kernel(x_ref, w_ref, h_ref):
    h_ref[0] = jnp.dot(x_ref[0], w_ref[...], preferred_element_type=jnp.float32)


def _gat_kernel(h_ref, hb_ref, as_ref, ad_ref, b_ref, mask_ref, out_ref, *,
                apply_elu):
    h = h_ref[0]
    hb = hb_ref[0]
    a_s = as_ref[...]
    a_d = ad_ref[...]
    asrc = jnp.sum(h * a_s, axis=1, keepdims=True)
    adst = jnp.sum(hb * a_d, axis=1)
    e = asrc + adst[None, :]
    e = jnp.where(e >= 0.0, e, 0.2 * e)
    mask = mask_ref[0] != 0
    em = jnp.max(jnp.where(mask, e, -jnp.inf), axis=0, keepdims=True)
    ex = jnp.where(mask, jnp.exp(e - em), 0.0)
    denom = jnp.sum(ex, axis=0, keepdims=True)
    alpha = ex / (denom + 1e-16)
    out = jax.lax.dot_general(alpha, h, (((0,), (0,)), ((), ())),
                              preferred_element_type=jnp.float32)
    out = out + b_ref[...]
    if apply_elu:
        out = jnp.where(out > 0.0, out, jnp.exp(jnp.minimum(out, 0.0)) - 1.0)
    out_ref[0] = out


def _build_mask(nodes, nodes_t, *, interpret=False):
    return pl.pallas_call(
        _mask_kernel,
        grid=(B_, PP // R_),
        in_specs=[
            pl.BlockSpec((1, R_, C_), lambda b, i: (b, i, 0)),
            pl.BlockSpec((1, C_, PP), lambda b, i: (b, 0, 0)),
        ],
        out_specs=pl.BlockSpec((1, R_, PP), lambda b, i: (b, i, 0)),
        out_shape=jax.ShapeDtypeStruct((B_, PP, PP), jnp.int8),
        interpret=interpret,
    )(nodes, nodes_t)


def _project(xn, w, *, interpret=False):
    c = xn.shape[-1]
    return pl.pallas_call(
        _proj_kernel,
        grid=(B_, PP // R_),
        in_specs=[
            pl.BlockSpec((1, R_, c), lambda b, i: (b, i, 0)),
            pl.BlockSpec((c, H_), lambda b, i: (0, 0)),
        ],
        out_specs=pl.BlockSpec((1, R_, H_), lambda b, i: (b, i, 0)),
        out_shape=jax.ShapeDtypeStruct((B_, PP, H_), jnp.float32),
        interpret=interpret,
    )(xn, w)


def _gat_layer(h, a_s, a_d, b, mask, *, apply_elu, interpret=False):
    return pl.pallas_call(
        functools.partial(_gat_kernel, apply_elu=apply_elu),
        grid=(B_, PP // CB),
        in_specs=[
            pl.BlockSpec((1, PP, H_), lambda b_, j: (b_, 0, 0)),
            pl.BlockSpec((1, CB, H_), lambda b_, j: (b_, j, 0)),
            pl.BlockSpec((1, H_), lambda b_, j: (0, 0)),
            pl.BlockSpec((1, H_), lambda b_, j: (0, 0)),
            pl.BlockSpec((1, H_), lambda b_, j: (0, 0)),
            pl.BlockSpec((1, PP, CB), lambda b_, j: (b_, 0, j)),
        ],
        out_specs=pl.BlockSpec((1, CB, H_), lambda b_, j: (b_, j, 0)),
        out_shape=jax.ShapeDtypeStruct((B_, PP, H_), jnp.float32),
        interpret=interpret,
    )(h, h, a_s, a_d, b, mask)


def _run(x, W1, att_src1, att_dst1, b1, W2, att_src2, att_dst2, b2,
         interpret=False):
    nodes = jnp.transpose(x, (0, 1, 3, 2)).reshape(B_, P_, C_)
    nodes_t = jnp.transpose(x, (0, 2, 1, 3)).reshape(B_, C_, P_)
    nodes = jnp.pad(nodes, ((0, 0), (0, PP - P_), (0, 0)))
    nodes_t = jnp.pad(nodes_t, ((0, 0), (0, 0), (0, PP - P_)))

    mask = _build_mask(nodes, nodes_t, interpret=interpret)

    h1 = _project(nodes, W1, interpret=interpret)
    out1 = _gat_layer(h1, att_src1.reshape(1, H_), att_dst1.reshape(1, H_),
                      b1.reshape(1, H_), mask, apply_elu=True,
                      interpret=interpret)
    h2 = _project(out1, W2, interpret=interpret)
    out2 = _gat_layer(h2, att_src2.reshape(1, H_), att_dst2.reshape(1, H_),
                      b2.reshape(1, H_), mask, apply_elu=False,
                      interpret=interpret)
    return out2[:, :P_, :].reshape(B_, N_, M_, H_)


def kernel(x, W1, att_src1, att_dst1, b1, W2, att_src2, att_dst2, b2):
    return _run(x, W1, att_src1, att_dst1, b1, W2, att_src2, att_dst2, b2)


# speedup vs baseline: 46.6615x; 1.6932x over previous
import functools

import jax
import jax.numpy as jnp
from jax.experimental import pallas as pl

B_ = 2
N_ = 20
C_ = 256
M_ = 250
K_ = 16
H_ = 256
P_ = N_ * M_
PP = 5120
R_ = 512
RM = 256
CB = 512
NCH = PP // 128
NLVL = 4
PAD_VAL = 1.0e4


def _nth_min(v, n):
    for _ in range(n - 1):
        m = jnp.min(v, axis=1, keepdims=True)
        v = jnp.where(v <= m, jnp.inf, v)
    return jnp.min(v, axis=1, keepdims=True)


def _mask_kernel(nodes_ref, nodes_t_ref, mask_ref):
    nb = nodes_ref[0]
    nt = nodes_t_ref[0]
    g = jnp.dot(nb, nt, preferred_element_type=jnp.float32)
    sqr = jnp.sum(nb * nb, axis=1, keepdims=True)
    sqc = jnp.sum(nt * nt, axis=0, keepdims=True)
    d2 = sqr + sqc - 2.0 * g
    r = nb.shape[0]

    chunks = [d2[:, k * 128:(k + 1) * 128] for k in range(NCH)]
    levels = []
    for lvl in range(NLVL):
        w = chunks[0]
        for k in range(1, NCH):
            w = jnp.minimum(w, chunks[k])
        levels.append(w)
        if lvl < NLVL - 1:
            chunks = [jnp.where(c <= w, jnp.inf, c) for c in chunks]
    cand = jnp.concatenate(levels, axis=1)
    t_hat = _nth_min(cand, K_ + 1)

    row = jax.lax.broadcasted_iota(jnp.int32, (r, 1), 0) + pl.program_id(1) * r
    row_ok = (row < P_).astype(jnp.float32)
    cnt = jnp.sum((d2 <= t_hat).astype(jnp.float32), axis=1, keepdims=True)
    badness = jnp.sum(jnp.abs(cnt - float(K_ + 1)) * row_ok)
    certified = badness == 0.0

    t = jax.lax.cond(certified,
                     lambda: t_hat,
                     lambda: _nth_min(d2, K_ + 1))
    mask_ref[0] = (d2 <= t).astype(jnp.float32) * row_ok


def _proj_kernel(x_ref, w_ref, as_ref, ad_ref, h_ref, asrc_ref, adst_ref):
    h = jnp.dot(x_ref[0], w_ref[...], preferred_element_type=jnp.float32)
    h_ref[0] = h
    asrc_ref[0] = jnp.sum(h * as_ref[...], axis=1, keepdims=True)
    adst_ref[0] = jnp.transpose(
        jnp.sum(h * ad_ref[...], axis=1, keepdims=True))


def _gat_kernel(h_ref, asrc_ref, adst_ref, b_ref, mask_ref, out_ref, *,
                apply_elu):
    h = h_ref[0]
    asrc = asrc_ref[0]
    adst = adst_ref[0]
    e = asrc + adst
    e = jnp.maximum(e, 0.2 * e)
    ex = jnp.exp(e) * mask_ref[0]
    denom = jnp.sum(ex, axis=0, keepdims=True)
    num = jax.lax.dot_general(ex, h, (((0,), (0,)), ((), ())),
                              preferred_element_type=jnp.float32)
    rec = jnp.transpose(1.0 / (denom + 1e-16))
    out = num * rec + b_ref[...]
    if apply_elu:
        out = jnp.where(out > 0.0, out, jnp.exp(jnp.minimum(out, 0.0)) - 1.0)
    out_ref[0] = out


def _build_mask(nodes, nodes_t, *, interpret=False):
    return pl.pallas_call(
        _mask_kernel,
        grid=(B_, PP // RM),
        in_specs=[
            pl.BlockSpec((1, RM, C_), lambda b, i: (b, i, 0)),
            pl.BlockSpec((1, C_, PP), lambda b, i: (b, 0, 0)),
        ],
        out_specs=pl.BlockSpec((1, RM, PP), lambda b, i: (b, i, 0)),
        out_shape=jax.ShapeDtypeStruct((B_, PP, PP), jnp.float32),
        interpret=interpret,
    )(nodes, nodes_t)


def _project(xn, w, a_s, a_d, *, interpret=False):
    c = xn.shape[-1]
    return pl.pallas_call(
        _proj_kernel,
        grid=(B_, PP // R_),
        in_specs=[
            pl.BlockSpec((1, R_, c), lambda b, i: (b, i, 0)),
            pl.BlockSpec((c, H_), lambda b, i: (0, 0)),
            pl.BlockSpec((1, H_), lambda b, i: (0, 0)),
            pl.BlockSpec((1, H_), lambda b, i: (0, 0)),
        ],
        out_specs=[
            pl.BlockSpec((1, R_, H_), lambda b, i: (b, i, 0)),
            pl.BlockSpec((1, R_, 1), lambda b, i: (b, i, 0)),
            pl.BlockSpec((1, 1, R_), lambda b, i: (b, 0, i)),
        ],
        out_shape=[
            jax.ShapeDtypeStruct((B_, PP, H_), jnp.float32),
            jax.ShapeDtypeStruct((B_, PP, 1), jnp.float32),
            jax.ShapeDtypeStruct((B_, 1, PP), jnp.float32),
        ],
        interpret=interpret,
    )(xn, w, a_s, a_d)


def _gat_layer(h, asrc, adst, b, mask, *, apply_elu, interpret=False):
    return pl.pallas_call(
        functools.partial(_gat_kernel, apply_elu=apply_elu),
        grid=(B_, PP // CB),
        in_specs=[
            pl.BlockSpec((1, PP, H_), lambda b_, j: (b_, 0, 0)),
            pl.BlockSpec((1, PP, 1), lambda b_, j: (b_, 0, 0)),
            pl.BlockSpec((1, 1, CB), lambda b_, j: (b_, 0, j)),
            pl.BlockSpec((1, H_), lambda b_, j: (0, 0)),
            pl.BlockSpec((1, PP, CB), lambda b_, j: (b_, 0, j)),
        ],
        out_specs=pl.BlockSpec((1, CB, H_), lambda b_, j: (b_, j, 0)),
        out_shape=jax.ShapeDtypeStruct((B_, PP, H_), jnp.float32),
        interpret=interpret,
    )(h, asrc, adst, b, mask)


def _run(x, W1, att_src1, att_dst1, b1, W2, att_src2, att_dst2, b2,
         interpret=False):
    nodes = jnp.transpose(x, (0, 1, 3, 2)).reshape(B_, P_, C_)
    nodes_t = jnp.transpose(x, (0, 2, 1, 3)).reshape(B_, C_, P_)
    nodes = jnp.pad(nodes, ((0, 0), (0, PP - P_), (0, 0)))
    nodes_t = jnp.pad(nodes_t, ((0, 0), (0, 0), (0, PP - P_)),
                      constant_values=PAD_VAL)

    mask = _build_mask(nodes, nodes_t, interpret=interpret)

    h1, asrc1, adst1 = _project(nodes, W1, att_src1.reshape(1, H_),
                                att_dst1.reshape(1, H_), interpret=interpret)
    out1 = _gat_layer(h1, asrc1, adst1, b1.reshape(1, H_), mask,
                      apply_elu=True, interpret=interpret)
    h2, asrc2, adst2 = _project(out1, W2, att_src2.reshape(1, H_),
                                att_dst2.reshape(1, H_), interpret=interpret)
    out2 = _gat_layer(h2, asrc2, adst2, b2.reshape(1, H_), mask,
                      apply_elu=False, interpret=interpret)
    return out2[:, :P_, :].reshape(B_, N_, M_, H_)


def kernel(x, W1, att_src1, att_dst1, b1, W2, att_src2, att_dst2, b2):
    return _run(x, W1, att_src1, att_dst1, b1, W2, att_src2, att_dst2, b2)


# speedup vs baseline: 48.6401x; 1.0424x over previous
import functools

import jax
import jax.numpy as jnp
from jax.experimental import pallas as pl

B_ = 2
N_ = 20
C_ = 256
M_ = 250
K_ = 16
H_ = 256
P_ = N_ * M_
PP = 5120
R_ = 512
RM = 512
CB = 512
NCH = PP // 128
NLVL = 4
PAD_VAL = 1.0e4


def _nth_min(v, n):
    for _ in range(n - 1):
        m = jnp.min(v, axis=1, keepdims=True)
        v = jnp.where(v <= m, jnp.inf, v)
    return jnp.min(v, axis=1, keepdims=True)


def _mask_kernel(nodes_ref, nodes_t_ref, mask_ref):
    nb = nodes_ref[0]
    nt = nodes_t_ref[0]
    g = jnp.dot(nb, nt, preferred_element_type=jnp.float32)
    sqr = jnp.sum(nb * nb, axis=1, keepdims=True)
    sqc = jnp.sum(nt * nt, axis=0, keepdims=True)
    d2 = sqr + sqc - 2.0 * g
    r = nb.shape[0]

    chunks = [d2[:, k * 128:(k + 1) * 128] for k in range(NCH)]
    levels = []
    for lvl in range(NLVL):
        w = chunks[0]
        for k in range(1, NCH):
            w = jnp.minimum(w, chunks[k])
        levels.append(w)
        if lvl < NLVL - 1:
            chunks = [jnp.where(c <= w, jnp.inf, c) for c in chunks]
    cand = jnp.concatenate(levels, axis=1)
    t_hat = _nth_min(cand, K_ + 1)

    row = jax.lax.broadcasted_iota(jnp.int32, (r, 1), 0) + pl.program_id(1) * r
    row_ok = (row < P_).astype(jnp.float32)
    maskf = (d2 <= t_hat).astype(jnp.float32) * row_ok
    cnt = jnp.sum(maskf, axis=1, keepdims=True)
    badness = jnp.sum(jnp.abs(cnt - float(K_ + 1)) * row_ok)
    mask_ref[0] = maskf.astype(jnp.bfloat16)

    @pl.when(badness != 0.0)
    def _fallback():
        t = _nth_min(d2, K_ + 1)
        mask_ref[0] = ((d2 <= t).astype(jnp.float32) * row_ok
                       ).astype(jnp.bfloat16)


def _proj_kernel(x_ref, w_ref, as_ref, ad_ref, h_ref, asrc_ref, adst_ref):
    h = jnp.dot(x_ref[0], w_ref[...], preferred_element_type=jnp.float32)
    h_ref[0] = h
    asrc_ref[0] = jnp.sum(h * as_ref[...], axis=1, keepdims=True)
    adst_ref[0] = jnp.transpose(
        jnp.sum(h * ad_ref[...], axis=1, keepdims=True))


def _gat_kernel(h_ref, asrc_ref, adst_ref, b_ref, mask_ref, out_ref, *,
                apply_elu):
    h = h_ref[0]
    asrc = asrc_ref[0]
    adst = adst_ref[0]
    e = asrc + adst
    e = jnp.maximum(e, 0.2 * e)
    ex = jnp.exp(e) * mask_ref[0].astype(jnp.float32)
    denom = jnp.sum(ex, axis=0, keepdims=True)
    num = jax.lax.dot_general(ex, h, (((0,), (0,)), ((), ())),
                              preferred_element_type=jnp.float32)
    rec = jnp.transpose(1.0 / (denom + 1e-16))
    out = num * rec + b_ref[...]
    if apply_elu:
        out = jnp.where(out > 0.0, out, jnp.exp(jnp.minimum(out, 0.0)) - 1.0)
    out_ref[0] = out


def _build_mask(nodes, nodes_t, *, interpret=False):
    return pl.pallas_call(
        _mask_kernel,
        grid=(B_, PP // RM),
        in_specs=[
            pl.BlockSpec((1, RM, C_), lambda b, i: (b, i, 0)),
            pl.BlockSpec((1, C_, PP), lambda b, i: (b, 0, 0)),
        ],
        out_specs=pl.BlockSpec((1, RM, PP), lambda b, i: (b, i, 0)),
        out_shape=jax.ShapeDtypeStruct((B_, PP, PP), jnp.bfloat16),
        interpret=interpret,
    )(nodes, nodes_t)


def _project(xn, w, a_s, a_d, *, interpret=False):
    c = xn.shape[-1]
    return pl.pallas_call(
        _proj_kernel,
        grid=(B_, PP // R_),
        in_specs=[
            pl.BlockSpec((1, R_, c), lambda b, i: (b, i, 0)),
            pl.BlockSpec((c, H_), lambda b, i: (0, 0)),
            pl.BlockSpec((1, H_), lambda b, i: (0, 0)),
            pl.BlockSpec((1, H_), lambda b, i: (0, 0)),
        ],
        out_specs=[
            pl.BlockSpec((1, R_, H_), lambda b, i: (b, i, 0)),
            pl.BlockSpec((1, R_, 1), lambda b, i: (b, i, 0)),
            pl.BlockSpec((1, 1, R_), lambda b, i: (b, 0, i)),
        ],
        out_shape=[
            jax.ShapeDtypeStruct((B_, PP, H_), jnp.float32),
            jax.ShapeDtypeStruct((B_, PP, 1), jnp.float32),
            jax.ShapeDtypeStruct((B_, 1, PP), jnp.float32),
        ],
        interpret=interpret,
    )(xn, w, a_s, a_d)


def _gat_layer(h, asrc, adst, b, mask, *, apply_elu, interpret=False):
    return pl.pallas_call(
        functools.partial(_gat_kernel, apply_elu=apply_elu),
        grid=(B_, PP // CB),
        in_specs=[
            pl.BlockSpec((1, PP, H_), lambda b_, j: (b_, 0, 0)),
            pl.BlockSpec((1, PP, 1), lambda b_, j: (b_, 0, 0)),
            pl.BlockSpec((1, 1, CB), lambda b_, j: (b_, 0, j)),
            pl.BlockSpec((1, H_), lambda b_, j: (0, 0)),
            pl.BlockSpec((1, PP, CB), lambda b_, j: (b_, 0, j)),
        ],
        out_specs=pl.BlockSpec((1, CB, H_), lambda b_, j: (b_, j, 0)),
        out_shape=jax.ShapeDtypeStruct((B_, PP, H_), jnp.float32),
        interpret=interpret,
    )(h, asrc, adst, b, mask)


def _run(x, W1, att_src1, att_dst1, b1, W2, att_src2, att_dst2, b2,
         interpret=False):
    nodes = jnp.transpose(x, (0, 1, 3, 2)).reshape(B_, P_, C_)
    nodes_t = jnp.transpose(x, (0, 2, 1, 3)).reshape(B_, C_, P_)
    nodes = jnp.pad(nodes, ((0, 0), (0, PP - P_), (0, 0)))
    nodes_t = jnp.pad(nodes_t, ((0, 0), (0, 0), (0, PP - P_)),
                      constant_values=PAD_VAL)

    mask = _build_mask(nodes, nodes_t, interpret=interpret)

    h1, asrc1, adst1 = _project(nodes, W1, att_src1.reshape(1, H_),
                                att_dst1.reshape(1, H_), interpret=interpret)
    out1 = _gat_layer(h1, asrc1, adst1, b1.reshape(1, H_), mask,
                      apply_elu=True, interpret=interpret)
    h2, asrc2, adst2 = _project(out1, W2, att_src2.reshape(1, H_),
                                att_dst2.reshape(1, H_), interpret=interpret)
    out2 = _gat_layer(h2, asrc2, adst2, b2.reshape(1, H_), mask,
                      apply_elu=False, interpret=interpret)
    return out2[:, :P_, :].reshape(B_, N_, M_, H_)


def kernel(x, W1, att_src1, att_dst1, b1, W2, att_src2, att_dst2, b2):
    return _run(x, W1, att_src1, att_dst1, b1, W2, att_src2, att_dst2, b2)


# speedup vs baseline: 60.7288x; 1.2485x over previous
import functools

import jax
import jax.numpy as jnp
from jax.experimental import pallas as pl

B_ = 2
N_ = 20
C_ = 256
M_ = 250
K_ = 16
H_ = 256
MB = 256
P_ = N_ * M_
PP = N_ * MB
R_ = 512
RM = 512
CB = 512
NCH = PP // 128
NLVL = 4
PAD_VAL = 1.0e4
CONTRACT_0 = (((0,), (0,)), ((), ()))


def _nt_kernel(x_ref, nt_ref):
    v = x_ref[0, 0]
    pad = jnp.full((C_, MB - M_), PAD_VAL, jnp.float32)
    nt_ref[0] = jnp.concatenate([v, pad], axis=1)


def _nth_min(v, n):
    for _ in range(n - 1):
        m = jnp.min(v, axis=1, keepdims=True)
        v = jnp.where(v <= m, jnp.inf, v)
    return jnp.min(v, axis=1, keepdims=True)


def _mask_kernel(ntb_ref, nt_ref, mask_ref):
    ntb = ntb_ref[0]
    nt = nt_ref[0]
    g = jax.lax.dot_general(ntb, nt, CONTRACT_0,
                            preferred_element_type=jnp.float32)
    sqc = jnp.sum(nt * nt, axis=0, keepdims=True)
    d2 = sqc - 2.0 * g

    chunks = [d2[:, k * 128:(k + 1) * 128] for k in range(NCH)]
    levels = []
    for lvl in range(NLVL):
        w = chunks[0]
        for k in range(1, NCH):
            w = jnp.minimum(w, chunks[k])
        levels.append(w)
        if lvl < NLVL - 1:
            chunks = [jnp.where(c <= w, jnp.inf, c) for c in chunks]
    cand = jnp.concatenate(levels, axis=1)
    t_hat = _nth_min(cand, K_ + 1)

    row = jax.lax.broadcasted_iota(jnp.int32, (RM, 1), 0)
    row_ok = (jax.lax.rem(row, MB) < M_).astype(jnp.float32)
    maskf = (d2 <= t_hat).astype(jnp.float32) * row_ok
    cnt = jnp.sum(maskf, axis=1, keepdims=True)
    badness = jnp.sum(jnp.abs(cnt - float(K_ + 1)) * row_ok)
    mask_ref[0] = maskf.astype(jnp.bfloat16)

    @pl.when(badness != 0.0)
    def _fallback():
        t = _nth_min(d2, K_ + 1)
        mask_ref[0] = ((d2 <= t).astype(jnp.float32) * row_ok
                       ).astype(jnp.bfloat16)


def _proj_t_kernel(ntb_ref, w_ref, as_ref, ad_ref, h_ref, asrc_ref, adst_ref):
    h = jax.lax.dot_general(ntb_ref[0], w_ref[...], CONTRACT_0,
                            preferred_element_type=jnp.float32)
    h_ref[0] = h.astype(jnp.bfloat16)
    asrc_ref[0] = jnp.sum(h * as_ref[...], axis=1, keepdims=True)
    adst_ref[0] = jnp.transpose(
        jnp.sum(h * ad_ref[...], axis=1, keepdims=True))


def _proj_kernel(x_ref, w_ref, as_ref, ad_ref, h_ref, asrc_ref, adst_ref):
    h = jnp.dot(x_ref[0], w_ref[...], preferred_element_type=jnp.float32)
    h_ref[0] = h.astype(jnp.bfloat16)
    asrc_ref[0] = jnp.sum(h * as_ref[...], axis=1, keepdims=True)
    adst_ref[0] = jnp.transpose(
        jnp.sum(h * ad_ref[...], axis=1, keepdims=True))


def _gat_kernel(h_ref, asrc_ref, adst_ref, b_ref, mask_ref, out_ref, *,
                apply_elu):
    h = h_ref[0]
    asrc = asrc_ref[0]
    adst = adst_ref[0]
    e = asrc + adst
    e = jnp.maximum(e, 0.2 * e)
    ex = jnp.exp(e) * mask_ref[0].astype(jnp.float32)
    denom = jnp.sum(ex, axis=0, keepdims=True)
    num = jax.lax.dot_general(ex.astype(jnp.bfloat16), h, CONTRACT_0,
                              preferred_element_type=jnp.float32)
    rec = jnp.transpose(1.0 / (denom + 1e-16))
    out = num * rec + b_ref[...]
    if apply_elu:
        out = jnp.where(out > 0.0, out, jnp.exp(jnp.minimum(out, 0.0)) - 1.0)
    out_ref[0] = out


def _to_nt(x, *, interpret=False):
    return pl.pallas_call(
        _nt_kernel,
        grid=(B_, N_),
        in_specs=[pl.BlockSpec((1, 1, C_, M_), lambda b, n: (b, n, 0, 0))],
        out_specs=pl.BlockSpec((1, C_, MB), lambda b, n: (b, 0, n)),
        out_shape=jax.ShapeDtypeStruct((B_, C_, PP), jnp.float32),
        interpret=interpret,
    )(x)


def _build_mask(nt, *, interpret=False):
    return pl.pallas_call(
        _mask_kernel,
        grid=(B_, PP // RM),
        in_specs=[
            pl.BlockSpec((1, C_, RM), lambda b, i: (b, 0, i)),
            pl.BlockSpec((1, C_, PP), lambda b, i: (b, 0, 0)),
        ],
        out_specs=pl.BlockSpec((1, RM, PP), lambda b, i: (b, i, 0)),
        out_shape=jax.ShapeDtypeStruct((B_, PP, PP), jnp.bfloat16),
        interpret=interpret,
    )(nt, nt)


def _project(xn, w, a_s, a_d, *, transposed, interpret=False):
    if transposed:
        body, spec = _proj_t_kernel, pl.BlockSpec((1, C_, R_),
                                                  lambda b, i: (b, 0, i))
    else:
        body, spec = _proj_kernel, pl.BlockSpec((1, R_, H_),
                                                lambda b, i: (b, i, 0))
    return pl.pallas_call(
        body,
        grid=(B_, PP // R_),
        in_specs=[
            spec,
            pl.BlockSpec((C_, H_), lambda b, i: (0, 0)),
            pl.BlockSpec((1, H_), lambda b, i: (0, 0)),
            pl.BlockSpec((1, H_), lambda b, i: (0, 0)),
        ],
        out_specs=[
            pl.BlockSpec((1, R_, H_), lambda b, i: (b, i, 0)),
            pl.BlockSpec((1, R_, 1), lambda b, i: (b, i, 0)),
            pl.BlockSpec((1, 1, R_), lambda b, i: (b, 0, i)),
        ],
        out_shape=[
            jax.ShapeDtypeStruct((B_, PP, H_), jnp.bfloat16),
            jax.ShapeDtypeStruct((B_, PP, 1), jnp.float32),
            jax.ShapeDtypeStruct((B_, 1, PP), jnp.float32),
        ],
        interpret=interpret,
    )(xn, w, a_s, a_d)


def _gat_layer(h, asrc, adst, b, mask, *, apply_elu, interpret=False):
    return pl.pallas_call(
        functools.partial(_gat_kernel, apply_elu=apply_elu),
        grid=(B_, PP // CB),
        in_specs=[
            pl.BlockSpec((1, PP, H_), lambda b_, j: (b_, 0, 0)),
            pl.BlockSpec((1, PP, 1), lambda b_, j: (b_, 0, 0)),
            pl.BlockSpec((1, 1, CB), lambda b_, j: (b_, 0, j)),
            pl.BlockSpec((1, H_), lambda b_, j: (0, 0)),
            pl.BlockSpec((1, PP, CB), lambda b_, j: (b_, 0, j)),
        ],
        out_specs=pl.BlockSpec((1, CB, H_), lambda b_, j: (b_, j, 0)),
        out_shape=jax.ShapeDtypeStruct((B_, PP, H_), jnp.float32),
        interpret=interpret,
    )(h, asrc, adst, b, mask)


def _run(x, W1, att_src1, att_dst1, b1, W2, att_src2, att_dst2, b2,
         interpret=False):
    nt = _to_nt(x, interpret=interpret)

    mask = _build_mask(nt, interpret=interpret)

    h1, asrc1, adst1 = _project(nt, W1, att_src1.reshape(1, H_),
                                att_dst1.reshape(1, H_), transposed=True,
                                interpret=interpret)
    out1 = _gat_layer(h1, asrc1, adst1, b1.reshape(1, H_), mask,
                      apply_elu=True, interpret=interpret)
    h2, asrc2, adst2 = _project(out1, W2, att_src2.reshape(1, H_),
                                att_dst2.reshape(1, H_), transposed=False,
                                interpret=interpret)
    out2 = _gat_layer(h2, asrc2, adst2, b2.reshape(1, H_), mask,
                      apply_elu=False, interpret=interpret)
    return out2.reshape(B_, N_, MB, H_)[:, :, :M_, :]


def kernel(x, W1, att_src1, att_dst1, b1, W2, att_src2, att_dst2, b2):
    return _run(x, W1, att_src1, att_dst1, b1, W2, att_src2, att_dst2, b2)
